# baseline (speedup 1.0000x reference)
import functools
import math

import jax
import jax.numpy as jnp
from jax import lax
from jax.experimental import pallas as pl
from jax.experimental.pallas import tpu as pltpu
from jax.experimental.pallas import tpu_sc as plsc

N = 2048
E = 65536
D = 256
EMB = 1024
H = 4
C = 64
EW_W = 16

NC = 2
NS = 16
NW = NC * NS
E_PER_W = E // NW
CH = 128
ROWS_PER_S = N // NS

TE = 2048
EA = 16384


def _pack_bf16(a):
    ab = a.astype(jnp.bfloat16)
    lo = lax.bitcast_convert_type(ab[:, :D // 2], jnp.uint16).astype(jnp.uint32)
    hi = lax.bitcast_convert_type(ab[:, D // 2:], jnp.uint16).astype(jnp.uint32)
    return lax.bitcast_convert_type(lo | (hi << 16), jnp.int32)


def _unpack_bf16(p32):
    lo = lax.bitcast_convert_type(p32 << 16, jnp.float32)
    hi = lax.bitcast_convert_type(p32 & jnp.int32(-65536), jnp.float32)
    return jnp.concatenate([lo, hi], axis=1)


def _ln(v, g, b, eps=1e-5):
    mu = jnp.mean(v, axis=-1, keepdims=True)
    var = jnp.mean((v - mu) ** 2, axis=-1, keepdims=True)
    return (v - mu) * jax.lax.rsqrt(var + eps) * g + b


def _q_body(x_ref, wq_ref, bq_ref, q_ref):
    q_ref[...] = (
        jnp.dot(x_ref[...], wq_ref[...], preferred_element_type=jnp.float32)
        + bq_ref[...]
    )


def _q_proj(x, Wq, bq2):
    return pl.pallas_call(
        _q_body,
        out_shape=jax.ShapeDtypeStruct((N, D), jnp.float32),
    )(x, Wq, bq2)


@functools.lru_cache(maxsize=None)
def _make_sc_gather(ne, ebase):
    mesh = plsc.VectorSubcoreMesh(core_axis_name="c", subcore_axis_name="s")
    epw = ne // NW
    nch = epw // CH

    @functools.partial(
        pl.kernel,
        out_type=(
            jax.ShapeDtypeStruct((ne, D // 2), jnp.int32),
            jax.ShapeDtypeStruct((ne, D // 2), jnp.int32),
        ),
        mesh=mesh,
        scratch_types=[
            pltpu.VMEM((epw,), jnp.int32),
            pltpu.VMEM((epw,), jnp.int32),
            pltpu.VMEM((CH, D // 2), jnp.int32),
            pltpu.VMEM((CH, D // 2), jnp.int32),
            pltpu.SemaphoreType.DMA,
            pltpu.SemaphoreType.DMA,
        ],
    )
    def _sc_gather(x_hbm, q_hbm, dst_hbm, src_hbm, xd_hbm, qs_hbm,
                   idxd_v, idxs_v, bufa, bufb, semg, sems):
        c = lax.axis_index("c")
        s = lax.axis_index("s")
        obase = (c * NS + s) * epw
        pltpu.sync_copy(dst_hbm.at[pl.ds(ebase + obase, epw)], idxd_v)
        pltpu.sync_copy(src_hbm.at[pl.ds(ebase + obase, epw)], idxs_v)

        def phase(tab, idx_v, out):
            pltpu.async_copy(tab.at[idx_v.at[pl.ds(0, CH)]], bufa, semg).wait()

            def body(j, carry):
                cc = 2 * j
                g = pltpu.async_copy(
                    tab.at[idx_v.at[pl.ds((cc + 1) * CH, CH)]], bufb, semg)
                st = pltpu.async_copy(
                    bufa, out.at[pl.ds(obase + cc * CH, CH)], sems)
                g.wait()
                st.wait()
                g2 = pltpu.async_copy(
                    tab.at[idx_v.at[pl.ds((cc + 2) * CH, CH)]], bufa, semg)
                st2 = pltpu.async_copy(
                    bufb, out.at[pl.ds(obase + (cc + 1) * CH, CH)], sems)
                g2.wait()
                st2.wait()
                return carry

            lax.fori_loop(0, nch // 2 - 1, body, 0)
            g = pltpu.async_copy(
                tab.at[idx_v.at[pl.ds((nch - 1) * CH, CH)]], bufb, semg)
            st = pltpu.async_copy(
                bufa, out.at[pl.ds(obase + (nch - 2) * CH, CH)], sems)
            g.wait()
            st.wait()
            pltpu.sync_copy(bufb, out.at[pl.ds(obase + (nch - 1) * CH, CH)])

        phase(x_hbm, idxd_v, xd_hbm)
        phase(q_hbm, idxs_v, qs_hbm)

    return _sc_gather


NB = 256


def _edge_body(*args):
    (src3, xd, qs, ef, wk, bk, wv, bv, we, be, we1, be1, we2, be2,
     g1, b1, g2, b2, o_init, den_init) = args[:20]
    rest = args[20:]
    if len(rest) == 4:
        _e2_prev, e2_out, o_out, den_out = rest
    else:
        e2_out, o_out, den_out = rest
    i = pl.program_id(0)
    z = _unpack_bf16(xd[...]) * ef[...]
    zb = z.astype(jnp.bfloat16)
    k = jnp.dot(zb, wk[...], preferred_element_type=jnp.float32) + bk[...]
    v = jnp.dot(zb, wv[...], preferred_element_type=jnp.float32) + bv[...]
    prod = _unpack_bf16(qs[...]) * k
    scale = 1.0 / math.sqrt(C)
    cidx = lax.broadcasted_iota(jnp.int32, (D, EW_W), 0) // C
    hidx = lax.broadcasted_iota(jnp.int32, (D, EW_W), 1)
    sel = jnp.where(cidx == hidx, scale, 0.0).astype(jnp.bfloat16)
    s16 = jnp.dot(prod.astype(jnp.bfloat16), sel,
                  preferred_element_type=jnp.float32)
    hmask = (lax.broadcasted_iota(jnp.int32, (1, EW_W), 1) < H).astype(
        jnp.float32)
    ewts_f = jnp.exp(s16) * hmask
    ewts = ewts_f.astype(jnp.bfloat16)
    ridx = lax.broadcasted_iota(jnp.int32, (EW_W, D), 0)
    cidx2 = lax.broadcasted_iota(jnp.int32, (EW_W, D), 1) // C
    expand = (ridx == cidx2).astype(jnp.bfloat16)
    ewfull = jnp.dot(ewts, expand, preferred_element_type=jnp.float32)
    wvals = (ewfull * v).astype(jnp.bfloat16)

    @pl.when(i == 0)
    def _init():
        o_out[...] = o_init[...]
        den_out[...] = den_init[...]

    srow = src3[0]
    for nb in range(N // NB):
        iota = lax.broadcasted_iota(jnp.int32, (NB, TE), 0) + nb * NB
        oh = (iota == srow).astype(jnp.bfloat16)
        o_out[nb * NB:(nb + 1) * NB, :] += jnp.dot(
            oh, wvals, preferred_element_type=jnp.float32)
        den_out[nb * NB:(nb + 1) * NB, :] += jnp.dot(
            oh, ewts, preferred_element_type=jnp.float32)

    eo = jnp.dot(zb, we[...], preferred_element_type=jnp.float32) + be[...]
    e1 = _ln(ef[...] + eo, g1[...], b1[...])
    h1 = (jnp.maximum(
        jnp.dot(e1.astype(jnp.bfloat16), we1[...],
                preferred_element_type=jnp.float32) + be1[...], 0.0,
    )).astype(jnp.bfloat16)
    e2 = e1 + jnp.dot(h1, we2[...],
                      preferred_element_type=jnp.float32) + be2[...]
    e2_out[...] = _ln(e2, g2[...], b2[...])


def _edge_call(src3, xd, qs, ef, Wk, bk2, Wv, bv2, We, be_2, We1, be12, We2,
               be22, g1, b1, g2, b2, o_init, den_init, e2_prev, off, n_steps):
    tile = lambda i: (i, 0)
    tile_off = lambda i: (i + off, 0)
    rep = lambda i: (0, 0)
    eb_off = pl.BlockSpec((TE, D), tile_off)
    ebp = pl.BlockSpec((TE, D // 2), tile)
    wspec = pl.BlockSpec((D, D), rep)
    bspec = pl.BlockSpec((1, D), rep)
    bspec_e = pl.BlockSpec((1, EMB), rep)
    in_specs = [
        pl.BlockSpec((1, 1, TE), lambda i: (i + off, 0, 0)),
        ebp, ebp, eb_off,
        wspec, bspec, wspec, bspec, wspec, bspec,
        pl.BlockSpec((D, EMB), rep), bspec_e,
        pl.BlockSpec((EMB, D), rep), bspec,
        bspec, bspec, bspec, bspec,
        pl.BlockSpec((N, D), rep),
        pl.BlockSpec((N, EW_W), rep),
    ]
    inputs = (src3, xd, qs, ef, Wk, bk2, Wv, bv2, We, be_2, We1, be12, We2,
              be22, g1, b1, g2, b2, o_init, den_init)
    aliases = {}
    if e2_prev is not None:
        in_specs.append(pl.BlockSpec((8, 128), rep))
        inputs = inputs + (e2_prev,)
        aliases = {20: 0}
    return pl.pallas_call(
        _edge_body,
        grid=(n_steps,),
        in_specs=in_specs,
        out_specs=[
            eb_off,
            pl.BlockSpec((N, D), rep),
            pl.BlockSpec((N, EW_W), rep),
        ],
        out_shape=[
            jax.ShapeDtypeStruct((E, D), jnp.float32),
            jax.ShapeDtypeStruct((N, D), jnp.float32),
            jax.ShapeDtypeStruct((N, EW_W), jnp.float32),
        ],
        input_output_aliases=aliases,
    )(*inputs)


def _node_body(x, o_in, den_in, wx1, bx1, wx2, bx2, g1, b1, g2, b2, x2_out):
    o = o_in[...]
    den = den_in[...]
    blocks = []
    for h in range(H):
        dh = den[:, h:h + 1]
        dh = jnp.where(dh > 0.0, dh, 1.0)
        blocks.append(o[:, h * C:(h + 1) * C] / dh)
    attn = jnp.concatenate(blocks, axis=1)
    x1 = _ln(x[...] + attn, g1[...], b1[...])
    h1 = jnp.maximum(
        jnp.dot(x1.astype(jnp.bfloat16), wx1[...],
                preferred_element_type=jnp.float32) + bx1[...], 0.0
    )
    x2 = x1 + jnp.dot(h1.astype(jnp.bfloat16), wx2[...],
                      preferred_element_type=jnp.float32) + bx2[...]
    x2_out[...] = _ln(x2, g2[...], b2[...])


def _node_call(x, o, den, Wx1, bx12, Wx2, bx22, g1, b1, g2, b2):
    return pl.pallas_call(
        _node_body,
        out_shape=jax.ShapeDtypeStruct((N, D), jnp.float32),
    )(x, o, den, Wx1, bx12, Wx2, bx22, g1, b1, g2, b2)


def kernel(x, edge_index, edge_feat, Wq, bq, Wk, bk, Wv, bv, We, be,
           Wx1, bx1, Wx2, bx2, We1, be1, We2, be2,
           ln1_g, ln1_b, ln2_g, ln2_b):
    src = edge_index[0]
    dst = edge_index[1]
    r = lambda b: b.reshape(1, -1)
    g1, b1 = r(ln1_g), r(ln1_b)
    g2, b2 = r(ln2_g), r(ln2_b)

    q = _q_proj(x, Wq, r(bq))
    xp, qp = _pack_bf16(x), _pack_bf16(q)
    bf = jnp.bfloat16
    src3 = src.reshape(E // TE, 1, TE)
    wargs = (Wk.astype(bf), r(bk), Wv.astype(bf), r(bv), We.astype(bf), r(be),
             We1.astype(bf), r(be1), We2.astype(bf), r(be2), g1, b1, g2, b2)
    zo = jnp.zeros((N, D), jnp.float32)
    zd = jnp.zeros((N, EW_W), jnp.float32)
    splits = (8192, 16384, E - 8192 - 16384)
    o_c, den_c, e2_c = zo, zd, None
    ebase = 0
    for ne in splits:
        xd_i, qs_i = _make_sc_gather(ne, ebase)(xp, qp, dst, src)
        e2_c, o_c, den_c = _edge_call(
            src3, xd_i, qs_i, edge_feat, *wargs, o_c, den_c, e2_c,
            ebase // TE, ne // TE)
        ebase += ne
    x2 = _node_call(x, o_c, den_c, Wx1.astype(bf), r(bx1), Wx2.astype(bf),
                    r(bx2), g1, b1, g2, b2)
    return (x2, e2_c)


# speedup vs baseline: 1.0160x; 1.0160x over previous
import functools
import math

import jax
import jax.numpy as jnp
from jax import lax
from jax.experimental import pallas as pl
from jax.experimental.pallas import tpu as pltpu
from jax.experimental.pallas import tpu_sc as plsc

N = 2048
E = 65536
D = 256
EMB = 1024
H = 4
C = 64
EW_W = 16

NC = 2
NS = 16
NW = NC * NS
E_PER_W = E // NW
CH = 128
ROWS_PER_S = N // NS

TE = 2048
EA = 16384


def _pack_bf16(a):
    ab = a.astype(jnp.bfloat16)
    lo = lax.bitcast_convert_type(ab[:, :D // 2], jnp.uint16).astype(jnp.uint32)
    hi = lax.bitcast_convert_type(ab[:, D // 2:], jnp.uint16).astype(jnp.uint32)
    return lax.bitcast_convert_type(lo | (hi << 16), jnp.int32)


def _unpack_bf16(p32):
    lo = lax.bitcast_convert_type(p32 << 16, jnp.float32)
    hi = lax.bitcast_convert_type(p32 & jnp.int32(-65536), jnp.float32)
    return jnp.concatenate([lo, hi], axis=1)


def _ln(v, g, b, eps=1e-5):
    mu = jnp.mean(v, axis=-1, keepdims=True)
    var = jnp.mean((v - mu) ** 2, axis=-1, keepdims=True)
    return (v - mu) * jax.lax.rsqrt(var + eps) * g + b


def _q_body(x_ref, wq_ref, bq_ref, q_ref):
    q_ref[...] = (
        jnp.dot(x_ref[...], wq_ref[...], preferred_element_type=jnp.float32)
        + bq_ref[...]
    )


def _q_proj(x, Wq, bq2):
    return pl.pallas_call(
        _q_body,
        out_shape=jax.ShapeDtypeStruct((N, D), jnp.float32),
    )(x, Wq, bq2)


@functools.lru_cache(maxsize=None)
def _make_sc_gather(ne, ebase):
    mesh = plsc.VectorSubcoreMesh(core_axis_name="c", subcore_axis_name="s")
    epw = ne // NW
    nch = epw // CH

    @functools.partial(
        pl.kernel,
        out_type=(
            jax.ShapeDtypeStruct((ne, D // 2), jnp.int32),
            jax.ShapeDtypeStruct((ne, D // 2), jnp.int32),
        ),
        mesh=mesh,
        scratch_types=[
            pltpu.VMEM((epw,), jnp.int32),
            pltpu.VMEM((epw,), jnp.int32),
            pltpu.VMEM((CH, D // 2), jnp.int32),
            pltpu.VMEM((CH, D // 2), jnp.int32),
            pltpu.SemaphoreType.DMA,
            pltpu.SemaphoreType.DMA,
        ],
    )
    def _sc_gather(x_hbm, q_hbm, dst_hbm, src_hbm, xd_hbm, qs_hbm,
                   idxd_v, idxs_v, bufa, bufb, semg, sems):
        c = lax.axis_index("c")
        s = lax.axis_index("s")
        obase = (c * NS + s) * epw
        pltpu.sync_copy(dst_hbm.at[pl.ds(ebase + obase, epw)], idxd_v)
        pltpu.sync_copy(src_hbm.at[pl.ds(ebase + obase, epw)], idxs_v)

        def phase(tab, idx_v, out):
            pltpu.async_copy(tab.at[idx_v.at[pl.ds(0, CH)]], bufa, semg).wait()

            def body(j, carry):
                cc = 2 * j
                g = pltpu.async_copy(
                    tab.at[idx_v.at[pl.ds((cc + 1) * CH, CH)]], bufb, semg)
                st = pltpu.async_copy(
                    bufa, out.at[pl.ds(obase + cc * CH, CH)], sems)
                g.wait()
                st.wait()
                g2 = pltpu.async_copy(
                    tab.at[idx_v.at[pl.ds((cc + 2) * CH, CH)]], bufa, semg)
                st2 = pltpu.async_copy(
                    bufb, out.at[pl.ds(obase + (cc + 1) * CH, CH)], sems)
                g2.wait()
                st2.wait()
                return carry

            lax.fori_loop(0, nch // 2 - 1, body, 0)
            g = pltpu.async_copy(
                tab.at[idx_v.at[pl.ds((nch - 1) * CH, CH)]], bufb, semg)
            st = pltpu.async_copy(
                bufa, out.at[pl.ds(obase + (nch - 2) * CH, CH)], sems)
            g.wait()
            st.wait()
            pltpu.sync_copy(bufb, out.at[pl.ds(obase + (nch - 1) * CH, CH)])

        phase(x_hbm, idxd_v, xd_hbm)
        phase(q_hbm, idxs_v, qs_hbm)

    return _sc_gather


NB = 256


def _edge_body(*args):
    (src3, xd, qs, ef, wk, bk, wv, bv, we, be, we1, be1, we2, be2,
     g1, b1, g2, b2, o_init, den_init) = args[:20]
    rest = args[20:]
    if len(rest) == 4:
        _e2_prev, e2_out, o_out, den_out = rest
    else:
        e2_out, o_out, den_out = rest
    i = pl.program_id(0)
    z = _unpack_bf16(xd[...]) * ef[...]
    zb = z.astype(jnp.bfloat16)
    k = jnp.dot(zb, wk[...], preferred_element_type=jnp.float32) + bk[...]
    v = jnp.dot(zb, wv[...], preferred_element_type=jnp.float32) + bv[...]
    prod = _unpack_bf16(qs[...]) * k
    scale = 1.0 / math.sqrt(C)
    cidx = lax.broadcasted_iota(jnp.int32, (D, EW_W), 0) // C
    hidx = lax.broadcasted_iota(jnp.int32, (D, EW_W), 1)
    sel = jnp.where(cidx == hidx, scale, 0.0).astype(jnp.bfloat16)
    s16 = jnp.dot(prod.astype(jnp.bfloat16), sel,
                  preferred_element_type=jnp.float32)
    hmask = (lax.broadcasted_iota(jnp.int32, (1, EW_W), 1) < H).astype(
        jnp.float32)
    ewts_f = jnp.exp(s16) * hmask
    ewts = ewts_f.astype(jnp.bfloat16)
    ridx = lax.broadcasted_iota(jnp.int32, (EW_W, D), 0)
    cidx2 = lax.broadcasted_iota(jnp.int32, (EW_W, D), 1) // C
    expand = (ridx == cidx2).astype(jnp.bfloat16)
    ewfull = jnp.dot(ewts, expand, preferred_element_type=jnp.float32)
    wvals = (ewfull * v).astype(jnp.bfloat16)

    @pl.when(i == 0)
    def _init():
        o_out[...] = o_init[...]
        den_out[...] = den_init[...]

    srow = src3[0]
    for nb in range(N // NB):
        iota = lax.broadcasted_iota(jnp.int32, (NB, TE), 0) + nb * NB
        oh = (iota == srow).astype(jnp.bfloat16)
        o_out[nb * NB:(nb + 1) * NB, :] += jnp.dot(
            oh, wvals, preferred_element_type=jnp.float32)
        den_out[nb * NB:(nb + 1) * NB, :] += jnp.dot(
            oh, ewts, preferred_element_type=jnp.float32)

    eo = jnp.dot(zb, we[...], preferred_element_type=jnp.float32) + be[...]
    e1 = _ln(ef[...] + eo, g1[...], b1[...])
    h1 = (jnp.maximum(
        jnp.dot(e1.astype(jnp.bfloat16), we1[...],
                preferred_element_type=jnp.float32) + be1[...], 0.0,
    )).astype(jnp.bfloat16)
    e2 = e1 + jnp.dot(h1, we2[...],
                      preferred_element_type=jnp.float32) + be2[...]
    e2_out[...] = _ln(e2, g2[...], b2[...])


def _edge_call(src3, xd, qs, ef, Wk, bk2, Wv, bv2, We, be_2, We1, be12, We2,
               be22, g1, b1, g2, b2, o_init, den_init, e2_prev, off, n_steps):
    tile = lambda i: (i, 0)
    tile_off = lambda i: (i + off, 0)
    rep = lambda i: (0, 0)
    eb_off = pl.BlockSpec((TE, D), tile_off)
    ebp = pl.BlockSpec((TE, D // 2), tile)
    wspec = pl.BlockSpec((D, D), rep)
    bspec = pl.BlockSpec((1, D), rep)
    bspec_e = pl.BlockSpec((1, EMB), rep)
    in_specs = [
        pl.BlockSpec((1, 1, TE), lambda i: (i + off, 0, 0)),
        ebp, ebp, eb_off,
        wspec, bspec, wspec, bspec, wspec, bspec,
        pl.BlockSpec((D, EMB), rep), bspec_e,
        pl.BlockSpec((EMB, D), rep), bspec,
        bspec, bspec, bspec, bspec,
        pl.BlockSpec((N, D), rep),
        pl.BlockSpec((N, EW_W), rep),
    ]
    inputs = (src3, xd, qs, ef, Wk, bk2, Wv, bv2, We, be_2, We1, be12, We2,
              be22, g1, b1, g2, b2, o_init, den_init)
    aliases = {}
    if e2_prev is not None:
        in_specs.append(pl.BlockSpec((8, 128), rep))
        inputs = inputs + (e2_prev,)
        aliases = {20: 0}
    return pl.pallas_call(
        _edge_body,
        grid=(n_steps,),
        in_specs=in_specs,
        out_specs=[
            eb_off,
            pl.BlockSpec((N, D), rep),
            pl.BlockSpec((N, EW_W), rep),
        ],
        out_shape=[
            jax.ShapeDtypeStruct((E, D), jnp.float32),
            jax.ShapeDtypeStruct((N, D), jnp.float32),
            jax.ShapeDtypeStruct((N, EW_W), jnp.float32),
        ],
        input_output_aliases=aliases,
    )(*inputs)


def _node_body(x, o_in, den_in, wx1, bx1, wx2, bx2, g1, b1, g2, b2, x2_out):
    o = o_in[...]
    den = den_in[...]
    blocks = []
    for h in range(H):
        dh = den[:, h:h + 1]
        dh = jnp.where(dh > 0.0, dh, 1.0)
        blocks.append(o[:, h * C:(h + 1) * C] / dh)
    attn = jnp.concatenate(blocks, axis=1)
    x1 = _ln(x[...] + attn, g1[...], b1[...])
    h1 = jnp.maximum(
        jnp.dot(x1, wx1[...], preferred_element_type=jnp.float32) + bx1[...], 0.0
    )
    x2 = x1 + jnp.dot(h1, wx2[...], preferred_element_type=jnp.float32) + bx2[...]
    x2_out[...] = _ln(x2, g2[...], b2[...])


def _node_call(x, o, den, Wx1, bx12, Wx2, bx22, g1, b1, g2, b2):
    return pl.pallas_call(
        _node_body,
        out_shape=jax.ShapeDtypeStruct((N, D), jnp.float32),
    )(x, o, den, Wx1, bx12, Wx2, bx22, g1, b1, g2, b2)


def kernel(x, edge_index, edge_feat, Wq, bq, Wk, bk, Wv, bv, We, be,
           Wx1, bx1, Wx2, bx2, We1, be1, We2, be2,
           ln1_g, ln1_b, ln2_g, ln2_b):
    src = edge_index[0]
    dst = edge_index[1]
    r = lambda b: b.reshape(1, -1)
    g1, b1 = r(ln1_g), r(ln1_b)
    g2, b2 = r(ln2_g), r(ln2_b)

    q = _q_proj(x, Wq, r(bq))
    xp, qp = _pack_bf16(x), _pack_bf16(q)
    bf = jnp.bfloat16
    src3 = src.reshape(E // TE, 1, TE)
    wargs = (Wk.astype(bf), r(bk), Wv.astype(bf), r(bv), We.astype(bf), r(be),
             We1.astype(bf), r(be1), We2.astype(bf), r(be2), g1, b1, g2, b2)
    zo = jnp.zeros((N, D), jnp.float32)
    zd = jnp.zeros((N, EW_W), jnp.float32)
    splits = (EA, E - EA)
    o_c, den_c, e2_c = zo, zd, None
    ebase = 0
    for ne in splits:
        xd_i, qs_i = _make_sc_gather(ne, ebase)(xp, qp, dst, src)
        e2_c, o_c, den_c = _edge_call(
            src3, xd_i, qs_i, edge_feat, *wargs, o_c, den_c, e2_c,
            ebase // TE, ne // TE)
        ebase += ne
    x2 = _node_call(x, o_c, den_c, Wx1, r(bx1), Wx2, r(bx2), g1, b1, g2, b2)
    return (x2, e2_c)


# speedup vs baseline: 1.0167x; 1.0007x over previous
import functools
import math

import jax
import jax.numpy as jnp
from jax import lax
from jax.experimental import pallas as pl
from jax.experimental.pallas import tpu as pltpu
from jax.experimental.pallas import tpu_sc as plsc

N = 2048
E = 65536
D = 256
EMB = 1024
H = 4
C = 64
EW_W = 16

NC = 2
NS = 16
NW = NC * NS
CH = 128

TE = 2048
EA = 16384


def _pack_bf16(a):
    ab = a.astype(jnp.bfloat16)
    lo = lax.bitcast_convert_type(ab[:, :D // 2], jnp.uint16).astype(jnp.uint32)
    hi = lax.bitcast_convert_type(ab[:, D // 2:], jnp.uint16).astype(jnp.uint32)
    return lax.bitcast_convert_type(lo | (hi << 16), jnp.int32)


def _unpack_bf16(p32):
    lo = lax.bitcast_convert_type(p32 << 16, jnp.float32)
    hi = lax.bitcast_convert_type(p32 & jnp.int32(-65536), jnp.float32)
    return jnp.concatenate([lo, hi], axis=1)


def _ln(v, g, b, eps=1e-5):
    mu = jnp.mean(v, axis=-1, keepdims=True)
    var = jnp.mean((v - mu) ** 2, axis=-1, keepdims=True)
    return (v - mu) * jax.lax.rsqrt(var + eps) * g + b


def _q_body(x_ref, wq_ref, bq_ref, q_ref):
    q_ref[...] = (
        jnp.dot(x_ref[...], wq_ref[...], preferred_element_type=jnp.float32)
        + bq_ref[...]
    )


def _q_proj(x, Wq, bq2):
    return pl.pallas_call(
        _q_body,
        out_shape=jax.ShapeDtypeStruct((N, D), jnp.float32),
    )(x, Wq, bq2)


@functools.lru_cache(maxsize=None)
def _make_sc_gather(ne, ebase):
    mesh = plsc.VectorSubcoreMesh(core_axis_name="c", subcore_axis_name="s")
    epw = ne // NW
    nch = epw // CH

    @functools.partial(
        pl.kernel,
        out_type=(
            jax.ShapeDtypeStruct((ne, D // 2), jnp.int32),
            jax.ShapeDtypeStruct((ne, D // 2), jnp.int32),
        ),
        mesh=mesh,
        scratch_types=[
            pltpu.VMEM((epw,), jnp.int32),
            pltpu.VMEM((epw,), jnp.int32),
            pltpu.VMEM((CH, D // 2), jnp.int32),
            pltpu.VMEM((CH, D // 2), jnp.int32),
            pltpu.SemaphoreType.DMA,
            pltpu.SemaphoreType.DMA,
        ],
    )
    def _sc_gather(x_hbm, q_hbm, dst_hbm, src_hbm, xd_hbm, qs_hbm,
                   idxd_v, idxs_v, bufa, bufb, semg, sems):
        c = lax.axis_index("c")
        s = lax.axis_index("s")
        obase = (c * NS + s) * epw
        pltpu.sync_copy(dst_hbm.at[pl.ds(ebase + obase, epw)], idxd_v)
        pltpu.sync_copy(src_hbm.at[pl.ds(ebase + obase, epw)], idxs_v)

        def phase(tab, idx_v, out):
            pltpu.async_copy(tab.at[idx_v.at[pl.ds(0, CH)]], bufa, semg).wait()

            def body(j, carry):
                cc = 2 * j
                g = pltpu.async_copy(
                    tab.at[idx_v.at[pl.ds((cc + 1) * CH, CH)]], bufb, semg)
                st = pltpu.async_copy(
                    bufa, out.at[pl.ds(obase + cc * CH, CH)], sems)
                g.wait()
                st.wait()
                g2 = pltpu.async_copy(
                    tab.at[idx_v.at[pl.ds((cc + 2) * CH, CH)]], bufa, semg)
                st2 = pltpu.async_copy(
                    bufb, out.at[pl.ds(obase + (cc + 1) * CH, CH)], sems)
                g2.wait()
                st2.wait()
                return carry

            lax.fori_loop(0, nch // 2 - 1, body, 0)
            g = pltpu.async_copy(
                tab.at[idx_v.at[pl.ds((nch - 1) * CH, CH)]], bufb, semg)
            st = pltpu.async_copy(
                bufa, out.at[pl.ds(obase + (nch - 2) * CH, CH)], sems)
            g.wait()
            st.wait()
            pltpu.sync_copy(bufb, out.at[pl.ds(obase + (nch - 1) * CH, CH)])

        phase(x_hbm, idxd_v, xd_hbm)
        phase(q_hbm, idxs_v, qs_hbm)

    return _sc_gather


NB = 256


def _edge_body(*args):
    (src3, xd, qs, ef, wk, bk, wv, bv, we, be, we1, be1, we2, be2,
     g1, b1, g2, b2, o_init, den_init) = args[:20]
    rest = args[20:]
    if len(rest) == 4:
        _e2_prev, e2_out, o_out, den_out = rest
    else:
        e2_out, o_out, den_out = rest
    i = pl.program_id(0)
    z = _unpack_bf16(xd[...]) * ef[...]
    zb = z.astype(jnp.bfloat16)
    k = jnp.dot(zb, wk[...], preferred_element_type=jnp.float32) + bk[...]
    v = jnp.dot(zb, wv[...], preferred_element_type=jnp.float32) + bv[...]
    prod = _unpack_bf16(qs[...]) * k
    scale = 1.0 / math.sqrt(C)
    cidx = lax.broadcasted_iota(jnp.int32, (D, EW_W), 0) // C
    hidx = lax.broadcasted_iota(jnp.int32, (D, EW_W), 1)
    sel = jnp.where(cidx == hidx, scale, 0.0).astype(jnp.bfloat16)
    s16 = jnp.dot(prod.astype(jnp.bfloat16), sel,
                  preferred_element_type=jnp.float32)
    hmask = (lax.broadcasted_iota(jnp.int32, (1, EW_W), 1) < H).astype(
        jnp.float32)
    ewts_f = jnp.exp(s16) * hmask
    ewts = ewts_f.astype(jnp.bfloat16)
    ridx = lax.broadcasted_iota(jnp.int32, (EW_W, D), 0)
    cidx2 = lax.broadcasted_iota(jnp.int32, (EW_W, D), 1) // C
    expand = (ridx == cidx2).astype(jnp.bfloat16)
    ewfull = jnp.dot(ewts, expand, preferred_element_type=jnp.float32)
    wvals = (ewfull * v).astype(jnp.bfloat16)

    @pl.when(i == 0)
    def _init():
        o_out[...] = o_init[...]
        den_out[...] = den_init[...]

    srow = src3[0]
    for nb in range(N // NB):
        iota = lax.broadcasted_iota(jnp.int32, (NB, TE), 0) + nb * NB
        oh = (iota == srow).astype(jnp.bfloat16)
        o_out[nb * NB:(nb + 1) * NB, :] += jnp.dot(
            oh, wvals, preferred_element_type=jnp.float32)
        den_out[nb * NB:(nb + 1) * NB, :] += jnp.dot(
            oh, ewts, preferred_element_type=jnp.float32)

    eo = jnp.dot(zb, we[...], preferred_element_type=jnp.float32) + be[...]
    e1 = _ln(ef[...] + eo, g1[...], b1[...])
    h1 = (jnp.maximum(
        jnp.dot(e1.astype(jnp.bfloat16), we1[...],
                preferred_element_type=jnp.float32) + be1[...], 0.0,
    )).astype(jnp.bfloat16)
    e2 = e1 + jnp.dot(h1, we2[...],
                      preferred_element_type=jnp.float32) + be2[...]
    e2_out[...] = _ln(e2, g2[...], b2[...])


def _edge_call(src3, xd, qs, ef, Wk, bk2, Wv, bv2, We, be_2, We1, be12, We2,
               be22, g1, b1, g2, b2, o_init, den_init, e2_prev, off, n_steps):
    tile = lambda i: (i, 0)
    tile_off = lambda i: (i + off, 0)
    rep = lambda i: (0, 0)
    eb_off = pl.BlockSpec((TE, D), tile_off)
    ebp = pl.BlockSpec((TE, D // 2), tile)
    wspec = pl.BlockSpec((D, D), rep)
    bspec = pl.BlockSpec((1, D), rep)
    bspec_e = pl.BlockSpec((1, EMB), rep)
    in_specs = [
        pl.BlockSpec((1, 1, TE), lambda i: (i + off, 0, 0)),
        ebp, ebp, eb_off,
        wspec, bspec, wspec, bspec, wspec, bspec,
        pl.BlockSpec((D, EMB), rep), bspec_e,
        pl.BlockSpec((EMB, D), rep), bspec,
        bspec, bspec, bspec, bspec,
        pl.BlockSpec((N, D), rep),
        pl.BlockSpec((N, EW_W), rep),
    ]
    inputs = (src3, xd, qs, ef, Wk, bk2, Wv, bv2, We, be_2, We1, be12, We2,
              be22, g1, b1, g2, b2, o_init, den_init)
    aliases = {}
    if e2_prev is not None:
        in_specs.append(pl.BlockSpec((8, 128), rep))
        inputs = inputs + (e2_prev,)
        aliases = {20: 0}
    return pl.pallas_call(
        _edge_body,
        grid=(n_steps,),
        in_specs=in_specs,
        out_specs=[
            eb_off,
            pl.BlockSpec((N, D), rep),
            pl.BlockSpec((N, EW_W), rep),
        ],
        out_shape=[
            jax.ShapeDtypeStruct((E, D), jnp.float32),
            jax.ShapeDtypeStruct((N, D), jnp.float32),
            jax.ShapeDtypeStruct((N, EW_W), jnp.float32),
        ],
        input_output_aliases=aliases,
    )(*inputs)


def _node_body(x, o_in, den_in, wx1, bx1, wx2, bx2, g1, b1, g2, b2, x2_out):
    o = o_in[...]
    den = den_in[...]
    blocks = []
    for h in range(H):
        dh = den[:, h:h + 1]
        dh = jnp.where(dh > 0.0, dh, 1.0)
        blocks.append(o[:, h * C:(h + 1) * C] / dh)
    attn = jnp.concatenate(blocks, axis=1)
    x1 = _ln(x[...] + attn, g1[...], b1[...])
    h1 = jnp.maximum(
        jnp.dot(x1, wx1[...], preferred_element_type=jnp.float32) + bx1[...], 0.0
    )
    x2 = x1 + jnp.dot(h1, wx2[...], preferred_element_type=jnp.float32) + bx2[...]
    x2_out[...] = _ln(x2, g2[...], b2[...])


def _node_call(x, o, den, Wx1, bx12, Wx2, bx22, g1, b1, g2, b2):
    return pl.pallas_call(
        _node_body,
        out_shape=jax.ShapeDtypeStruct((N, D), jnp.float32),
    )(x, o, den, Wx1, bx12, Wx2, bx22, g1, b1, g2, b2)


def kernel(x, edge_index, edge_feat, Wq, bq, Wk, bk, Wv, bv, We, be,
           Wx1, bx1, Wx2, bx2, We1, be1, We2, be2,
           ln1_g, ln1_b, ln2_g, ln2_b):
    src = edge_index[0]
    dst = edge_index[1]
    r = lambda b: b.reshape(1, -1)
    g1, b1 = r(ln1_g), r(ln1_b)
    g2, b2 = r(ln2_g), r(ln2_b)

    q = _q_proj(x, Wq, r(bq))
    xp, qp = _pack_bf16(x), _pack_bf16(q)
    bf = jnp.bfloat16
    src3 = src.reshape(E // TE, 1, TE)
    wargs = (Wk.astype(bf), r(bk), Wv.astype(bf), r(bv), We.astype(bf), r(be),
             We1.astype(bf), r(be1), We2.astype(bf), r(be2), g1, b1, g2, b2)
    zo = jnp.zeros((N, D), jnp.float32)
    zd = jnp.zeros((N, EW_W), jnp.float32)
    splits = (EA, E - EA)
    o_c, den_c, e2_c = zo, zd, None
    ebase = 0
    for ne in splits:
        xd_i, qs_i = _make_sc_gather(ne, ebase)(xp, qp, dst, src)
        e2_c, o_c, den_c = _edge_call(
            src3, xd_i, qs_i, edge_feat, *wargs, o_c, den_c, e2_c,
            ebase // TE, ne // TE)
        ebase += ne
    x2 = _node_call(x, o_c, den_c, Wx1, r(bx1), Wx2, r(bx2), g1, b1, g2, b2)
    return (x2, e2_c)
